# Initial kernel scaffold; baseline (speedup 1.0000x reference)
import jax
import jax.numpy as jnp
from jax.experimental import pallas as pl


def kernel(location, table):
    raise NotImplementedError("write your pallas kernel here")


# speedup vs baseline: 2.7857x; 2.7857x over previous
import functools

import jax
import jax.numpy as jnp
from jax import lax
from jax.experimental import pallas as pl
from jax.experimental.pallas import tpu as pltpu
from jax.experimental.pallas import tpu_sc as plsc

CHUNK = 128


@functools.partial(jax.jit, static_argnames=("B", "D", "rows_per_w", "num_cores"))
def _sc_embedding_gather(idx2d, table, *, B, D, rows_per_w, num_cores):
    mesh = plsc.VectorSubcoreMesh(core_axis_name="c", subcore_axis_name="s")

    @functools.partial(
        pl.kernel,
        mesh=mesh,
        out_type=jax.ShapeDtypeStruct((B, D), jnp.float32),
        compiler_params=pltpu.CompilerParams(use_tc_tiling_on_sc=False),
        scratch_types=[
            pltpu.VMEM((rows_per_w, CHUNK), jnp.int32),
            pltpu.VMEM((CHUNK, D), jnp.float32),
            pltpu.SemaphoreType.DMA,
        ],
    )
    def k(idx_hbm, table_hbm, out_hbm, idx_v, rows_v, sem):
        wid = lax.axis_index("s") * num_cores + lax.axis_index("c")
        row_base = wid * rows_per_w
        pltpu.sync_copy(idx_hbm.at[pl.ds(row_base, rows_per_w)], idx_v)

        def body(j, carry):
            pltpu.async_copy(table_hbm.at[idx_v.at[j]], rows_v, sem).wait()
            out_off = (row_base + j) * CHUNK
            pltpu.sync_copy(rows_v, out_hbm.at[pl.ds(out_off, CHUNK)])
            return carry

        lax.fori_loop(0, rows_per_w, body, 0)

    return k(idx2d, table)


def kernel(location, table):
    batch, hist = location.shape
    vocab, D = table.shape
    B = batch * hist
    info = plsc.get_sparse_core_info()
    nw = info.num_cores * info.num_subcores
    n_chunks = B // CHUNK
    assert B % CHUNK == 0 and n_chunks % nw == 0
    idx2d = location.astype(jnp.int32).reshape(n_chunks, CHUNK)
    out = _sc_embedding_gather(
        idx2d,
        table.astype(jnp.float32),
        B=B,
        D=D,
        rows_per_w=n_chunks // nw,
        num_cores=info.num_cores,
    )
    return out.reshape(batch, hist, D)


# speedup vs baseline: 2.7898x; 1.0015x over previous
import functools

import jax
import jax.numpy as jnp
from jax import lax
from jax.experimental import pallas as pl
from jax.experimental.pallas import tpu as pltpu
from jax.experimental.pallas import tpu_sc as plsc

CHUNK = 128
NBUF = 4


@functools.partial(jax.jit, static_argnames=("B", "D", "rows_per_w", "num_cores"))
def _sc_embedding_gather(idx2d, table, *, B, D, rows_per_w, num_cores):
    mesh = plsc.VectorSubcoreMesh(core_axis_name="c", subcore_axis_name="s")
    n_groups = rows_per_w // NBUF
    assert rows_per_w % NBUF == 0

    @functools.partial(
        pl.kernel,
        mesh=mesh,
        out_type=jax.ShapeDtypeStruct((B, D), jnp.float32),
        compiler_params=pltpu.CompilerParams(use_tc_tiling_on_sc=False),
        scratch_types=[
            pltpu.VMEM((rows_per_w, CHUNK), jnp.int32),
            pltpu.VMEM((NBUF, CHUNK, D), jnp.float32),
            pltpu.SemaphoreType.DMA((NBUF,)),
            pltpu.SemaphoreType.DMA((NBUF,)),
        ],
    )
    def k(idx_hbm, table_hbm, out_hbm, idx_v, rows_v, gsem, wsem):
        wid = lax.axis_index("s") * num_cores + lax.axis_index("c")
        row_base = wid * rows_per_w
        pltpu.sync_copy(idx_hbm.at[pl.ds(row_base, rows_per_w)], idx_v)

        def start_gather(j, b):
            pltpu.async_copy(table_hbm.at[idx_v.at[j]], rows_v.at[b], gsem.at[b])

        def wait_gather(j, b):
            pltpu.make_async_copy(
                table_hbm.at[idx_v.at[j]], rows_v.at[b], gsem.at[b]
            ).wait()

        def start_write(j, b):
            pltpu.async_copy(
                rows_v.at[b], out_hbm.at[pl.ds((row_base + j) * CHUNK, CHUNK)],
                wsem.at[b],
            )

        def wait_write(b):
            pltpu.make_async_copy(
                rows_v.at[b], out_hbm.at[pl.ds(0, CHUNK)], wsem.at[b]
            ).wait()

        for b in range(NBUF):
            start_gather(b, b)

        def body(g, carry):
            j0 = g * NBUF
            for b in range(NBUF):
                wait_gather(j0 + b, b)
                start_write(j0 + b, b)

            @pl.when(g + 1 < n_groups)
            def _():
                for b in range(NBUF):
                    wait_write(b)
                    start_gather(j0 + NBUF + b, b)

            return carry

        lax.fori_loop(0, n_groups, body, 0)

        for b in range(NBUF):
            wait_write(b)

    return k(idx2d, table)


def kernel(location, table):
    batch, hist = location.shape
    vocab, D = table.shape
    B = batch * hist
    info = plsc.get_sparse_core_info()
    nw = info.num_cores * info.num_subcores
    n_chunks = B // CHUNK
    assert B % CHUNK == 0 and n_chunks % nw == 0
    idx2d = location.astype(jnp.int32).reshape(n_chunks, CHUNK)
    out = _sc_embedding_gather(
        idx2d,
        table.astype(jnp.float32),
        B=B,
        D=D,
        rows_per_w=n_chunks // nw,
        num_cores=info.num_cores,
    )
    return out.reshape(batch, hist, D)


# speedup vs baseline: 2.7991x; 1.0033x over previous
import functools

import jax
import jax.numpy as jnp
from jax import lax
from jax.experimental import pallas as pl
from jax.experimental.pallas import tpu as pltpu
from jax.experimental.pallas import tpu_sc as plsc

NBUF = 4


@functools.partial(
    jax.jit, static_argnames=("batch", "hist", "D", "num_cores", "num_subcores")
)
def _sc_embedding_gather(location, table, *, batch, hist, D, num_cores, num_subcores):
    mesh = plsc.VectorSubcoreMesh(core_axis_name="c", subcore_axis_name="s")
    num_workers = num_cores * num_subcores
    rows_per_w = batch // num_workers
    n_groups = rows_per_w // NBUF
    assert rows_per_w % NBUF == 0

    @functools.partial(
        pl.kernel,
        mesh=mesh,
        out_type=jax.ShapeDtypeStruct((batch, hist, D), jnp.float32),
        compiler_params=pltpu.CompilerParams(use_tc_tiling_on_sc=False),
        scratch_types=[
            pltpu.VMEM((rows_per_w, hist), jnp.int32),
            pltpu.VMEM((NBUF, hist, D), jnp.float32),
            pltpu.SemaphoreType.DMA((NBUF,)),
            pltpu.SemaphoreType.DMA((NBUF,)),
        ],
    )
    def k(idx_hbm, table_hbm, out_hbm, idx_v, rows_v, gsem, wsem):
        wid = lax.axis_index("s") * num_cores + lax.axis_index("c")
        row_base = wid * rows_per_w
        pltpu.sync_copy(idx_hbm.at[pl.ds(row_base, rows_per_w)], idx_v)

        def start_gather(i, b):
            pltpu.async_copy(table_hbm.at[idx_v.at[i]], rows_v.at[b], gsem.at[b])

        def wait_gather(i, b):
            pltpu.make_async_copy(
                table_hbm.at[idx_v.at[i]], rows_v.at[b], gsem.at[b]
            ).wait()

        def start_write(i, b):
            pltpu.async_copy(rows_v.at[b], out_hbm.at[row_base + i], wsem.at[b])

        def wait_write(b):
            pltpu.make_async_copy(
                rows_v.at[b], out_hbm.at[0], wsem.at[b]
            ).wait()

        for b in range(NBUF):
            start_gather(b, b)

        def body(g, carry):
            i0 = g * NBUF
            for b in range(NBUF):
                wait_gather(i0 + b, b)
                start_write(i0 + b, b)

            @pl.when(g + 1 < n_groups)
            def _():
                for b in range(NBUF):
                    wait_write(b)
                    start_gather(i0 + NBUF + b, b)

            return carry

        lax.fori_loop(0, n_groups, body, 0)

        for b in range(NBUF):
            wait_write(b)

    return k(location, table)


def kernel(location, table):
    batch, hist = location.shape
    vocab, D = table.shape
    info = plsc.get_sparse_core_info()
    assert batch % (info.num_cores * info.num_subcores * NBUF) == 0
    return _sc_embedding_gather(
        location.astype(jnp.int32),
        table.astype(jnp.float32),
        batch=batch,
        hist=hist,
        D=D,
        num_cores=info.num_cores,
        num_subcores=info.num_subcores,
    )


# speedup vs baseline: 4.3423x; 1.5513x over previous
import functools

import jax
import jax.numpy as jnp
from jax import lax
from jax.experimental import pallas as pl
from jax.experimental.pallas import tpu as pltpu
from jax.experimental.pallas import tpu_sc as plsc

NBUF = 2
LANE = 16
RUNROLL = 8


@functools.partial(
    jax.jit, static_argnames=("batch", "hist", "D", "vocab", "num_cores", "num_subcores")
)
def _sc_embedding_gather(idx_flat, table, *, batch, hist, D, vocab, num_cores, num_subcores):
    mesh = plsc.VectorSubcoreMesh(core_axis_name="c", subcore_axis_name="s")
    num_workers = num_cores * num_subcores
    rows_per_w = batch // num_workers
    idx_per_w = rows_per_w * hist
    n_vec = D // LANE
    n_rgrp = hist // RUNROLL
    assert hist % RUNROLL == 0 and D % LANE == 0

    @functools.partial(
        pl.kernel,
        mesh=mesh,
        out_type=jax.ShapeDtypeStruct((batch, hist, D), jnp.float32),
        scratch_types=[
            pltpu.VMEM((idx_per_w + LANE,), jnp.int32),
            pltpu.VMEM((vocab, D), jnp.float32),
            pltpu.VMEM((NBUF, hist, D), jnp.float32),
            pltpu.SemaphoreType.DMA((NBUF,)),
        ],
    )
    def k(idx_hbm, table_hbm, out_hbm, idx_v, table_v, rows_v, wsem):
        wid = lax.axis_index("s") * num_cores + lax.axis_index("c")
        row_base = wid * rows_per_w
        pltpu.sync_copy(table_hbm, table_v)
        pltpu.sync_copy(
            idx_hbm.at[pl.ds(row_base * hist, idx_per_w)],
            idx_v.at[pl.ds(0, idx_per_w)],
        )

        def fill_rows(i, b, r0, nrows):
            idx16 = idx_v[pl.ds(i * hist + r0, LANE)]
            for u in range(nrows):
                t = idx16[u]
                for c in range(n_vec):
                    rows_v[b, r0 + u, pl.ds(c * LANE, LANE)] = table_v[
                        t, pl.ds(c * LANE, LANE)
                    ]

        def fill_row(i, b):
            def rgrp(g, carry):
                fill_rows(i, b, g * LANE, LANE)
                return carry

            lax.fori_loop(0, hist // LANE, rgrp, 0)
            tail0 = (hist // LANE) * LANE
            if hist - tail0:
                fill_rows(i, b, tail0, hist - tail0)

        def start_write(i, b):
            pltpu.async_copy(rows_v.at[b], out_hbm.at[row_base + i], wsem.at[b])

        def wait_write(b):
            pltpu.make_async_copy(rows_v.at[b], out_hbm.at[0], wsem.at[b]).wait()

        for b in range(NBUF):
            fill_row(b, b)
            start_write(b, b)

        def body(i, carry):
            b = lax.rem(i, NBUF)
            wait_write(b)
            fill_row(i, b)
            start_write(i, b)
            return carry

        lax.fori_loop(NBUF, rows_per_w, body, 0)

        for b in range(NBUF):
            wait_write(b)

    return k(idx_flat, table)


def kernel(location, table):
    batch, hist = location.shape
    vocab, D = table.shape
    info = plsc.get_sparse_core_info()
    assert batch % (info.num_cores * info.num_subcores) == 0
    return _sc_embedding_gather(
        location.astype(jnp.int32).reshape(-1),
        table.astype(jnp.float32),
        batch=batch,
        hist=hist,
        D=D,
        vocab=vocab,
        num_cores=info.num_cores,
        num_subcores=info.num_subcores,
    )
